# baseline (speedup 1.0000x reference)
import functools

import jax
import jax.numpy as jnp
from jax import lax
from jax.experimental import pallas as pl
from jax.experimental.pallas import tpu as pltpu
from jax.experimental.pallas import tpu_sc as plsc

_LANES = 16
_HALF = 256
_IW = 128


@functools.cache
def _build_sampler(B, V, D):
    info = plsc.get_sparse_core_info()
    nc, ns = info.num_cores, info.num_subcores
    nw = nc * ns
    assert B % (8 * nw) == 0 and D % _LANES == 0
    b_per_w = B // nw
    n_steps = b_per_w // _HALF
    n_streams = _HALF * D // _IW
    mesh = plsc.VectorSubcoreMesh(core_axis_name="c", subcore_axis_name="s")

    @functools.partial(
        pl.kernel,
        mesh=mesh,
        out_type=jax.ShapeDtypeStruct((D, B), jnp.float32),
        compiler_params=pltpu.CompilerParams(use_tc_tiling_on_sc=False),
        scratch_types=[
            pltpu.VMEM((b_per_w,), jnp.int32),
            pltpu.VMEM((n_streams, _IW), jnp.int32),
            pltpu.VMEM((D * _HALF,), jnp.float32),
            pltpu.VMEM((D * _HALF,), jnp.float32),
            pltpu.VMEM((D, _HALF), jnp.float32),
            pltpu.SemaphoreType.DMA,
            pltpu.SemaphoreType.DMA,
            pltpu.SemaphoreType.DMA,
        ],
    )
    def sampler(y_hbm, muT_hbm, sgT_hbm, epsT_hbm, outT_hbm,
                idx_v, fidx_v, mu_v, sg_v, ep_v, sem_mu, sem_sg, sem_ep):
        wid = lax.axis_index("s") * nc + lax.axis_index("c")
        base = wid * b_per_w

        pltpu.sync_copy(y_hbm.at[pl.ds(base, b_per_w)], idx_v)

        def step(h, carry):
            col = base + h * _HALF
            cp_ep = pltpu.async_copy(
                epsT_hbm.at[:, pl.ds(col, _HALF)], ep_v, sem_ep)

            def build(g, carry2):
                yv = idx_v[pl.ds(h * _HALF + g * _LANES, _LANES)]
                for c in range(D):
                    pos = c * _HALF + g * _LANES
                    fidx_v[pos // _IW, pl.ds(pos % _IW, _LANES)] = (
                        yv + c * V)
                return carry2

            lax.fori_loop(0, _HALF // _LANES, build, 0)

            def fire(j, carry2):
                pltpu.async_copy(
                    muT_hbm.at[fidx_v.at[j]],
                    mu_v.at[pl.ds(j * _IW, _IW)], sem_mu)
                pltpu.async_copy(
                    sgT_hbm.at[fidx_v.at[j]],
                    sg_v.at[pl.ds(j * _IW, _IW)], sem_sg)
                return carry2

            lax.fori_loop(0, n_streams, fire, 0)

            def drain(j, carry2):
                pltpu.make_async_copy(
                    muT_hbm.at[fidx_v.at[j]],
                    mu_v.at[pl.ds(j * _IW, _IW)], sem_mu).wait()
                pltpu.make_async_copy(
                    sgT_hbm.at[fidx_v.at[j]],
                    sg_v.at[pl.ds(j * _IW, _IW)], sem_sg).wait()
                return carry2

            lax.fori_loop(0, n_streams, drain, 0)
            cp_ep.wait()

            def fma(c, carry2):
                for k in range(_HALF // _LANES):
                    sl = pl.ds(c * _HALF + k * _LANES, _LANES)
                    sl2 = pl.ds(k * _LANES, _LANES)
                    ep_v[c, sl2] = mu_v[sl] + sg_v[sl] * ep_v[c, sl2]
                return carry2

            lax.fori_loop(0, D, fma, 0)
            pltpu.sync_copy(ep_v, outT_hbm.at[:, pl.ds(col, _HALF)])
            return carry

        lax.fori_loop(0, n_steps, step, 0)

    return sampler


def kernel(y, mu, sigma):
    B = y.shape[0]
    V, D = mu.shape
    eps = jax.random.normal(jax.random.key(42), (B, D), dtype=mu.dtype)
    outT = _build_sampler(B, V, D)(
        y, mu.T.reshape(-1), sigma.T.reshape(-1), eps.T)
    return outT.T


# speedup vs baseline: 18.1684x; 18.1684x over previous
import functools

import jax
import jax.numpy as jnp
from jax import lax
from jax.experimental import pallas as pl
from jax.experimental.pallas import tpu as pltpu
from jax.experimental.pallas import tpu_sc as plsc

_LANES = 16
_TW = 128


@functools.cache
def _build_sampler(B, V, D):
    info = plsc.get_sparse_core_info()
    nc, ns = info.num_cores, info.num_subcores
    nw = nc * ns
    assert B % (8 * nw) == 0 and D % _LANES == 0
    b_per_w = B // nw
    tail = V % _TW
    vmain = V - tail
    tc_max = vmain - _TW
    mesh = plsc.VectorSubcoreMesh(core_axis_name="c", subcore_axis_name="s")

    @functools.partial(
        pl.kernel,
        mesh=mesh,
        out_type=jax.ShapeDtypeStruct((B, D), jnp.float32),
        compiler_params=pltpu.CompilerParams(needs_layout_passes=False),
        scratch_types=[
            pltpu.VMEM((b_per_w,), jnp.int32),
            pltpu.VMEM((D, b_per_w), jnp.float32),
            pltpu.VMEM((2, D, _TW), jnp.float32),
            pltpu.VMEM((2, D, _TW), jnp.float32),
            pltpu.VMEM((D, V % _TW), jnp.float32),
            pltpu.VMEM((D, V % _TW), jnp.float32),
            pltpu.VMEM((_LANES, D), jnp.float32),
            pltpu.SemaphoreType.DMA,
            pltpu.SemaphoreType.DMA,
            pltpu.SemaphoreType.DMA,
            pltpu.SemaphoreType.DMA,
        ],
    )
    def sampler(y_hbm, muT_hbm, sgT_hbm, epsT_hbm, tmu_hbm, tsg_hbm,
                out_hbm, idx_v, ep_v, mub_v, sgb_v, tmu_v, tsg_v, rows_v,
                sem_mu0, sem_mu1, sem_sg0, sem_sg1):
        sems = ((sem_mu0, sem_sg0), (sem_mu1, sem_sg1))
        wid = lax.axis_index("s") * nc + lax.axis_index("c")
        base = wid * b_per_w

        pltpu.sync_copy(y_hbm.at[pl.ds(base, b_per_w)], idx_v)
        pltpu.sync_copy(epsT_hbm.at[:, pl.ds(base, b_per_w)], ep_v)
        pltpu.sync_copy(tmu_hbm, tmu_v)
        pltpu.sync_copy(tsg_hbm, tsg_v)

        iotas = [
            lax.iota(jnp.int32, _LANES) + k * _LANES
            for k in range(D // _LANES)
        ]

        def issue(r, slot):
            tc = pl.multiple_of(
                jnp.minimum(r // _TW, tc_max // _TW) * _TW, _TW)
            pltpu.async_copy(
                muT_hbm.at[:, pl.ds(tc, _TW)], mub_v.at[slot], sems[slot][0])
            pltpu.async_copy(
                sgT_hbm.at[:, pl.ds(tc, _TW)], sgb_v.at[slot], sems[slot][1])

        def drain(slot):
            pltpu.make_async_copy(
                muT_hbm.at[:, pl.ds(0, _TW)], mub_v.at[slot],
                sems[slot][0]).wait()
            pltpu.make_async_copy(
                sgT_hbm.at[:, pl.ds(0, _TW)], sgb_v.at[slot],
                sems[slot][1]).wait()

        def extract(r, i, row, slot):
            tc = jnp.minimum(r // _TW, tc_max // _TW) * _TW
            rmod = jnp.minimum(r - tc, _TW - 1)
            use_tail = r >= vmain
            jt = jnp.clip(r - vmain, 0, tail - 1)
            jv = jnp.full((_LANES,), rmod, dtype=jnp.int32)
            jtv = jnp.full((_LANES,), jt, dtype=jnp.int32)
            pv = jnp.full((_LANES,), i, dtype=jnp.int32)
            for k in range(D // _LANES):
                cidx = iotas[k]
                mval = plsc.load_gather(mub_v.at[slot], [cidx, jv])
                sval = plsc.load_gather(sgb_v.at[slot], [cidx, jv])
                mtail = plsc.load_gather(tmu_v, [cidx, jtv])
                stail = plsc.load_gather(tsg_v, [cidx, jtv])
                mval = jnp.where(use_tail, mtail, mval)
                sval = jnp.where(use_tail, stail, sval)
                eval_ = plsc.load_gather(ep_v, [cidx, pv])
                rows_v[row, pl.ds(k * _LANES, _LANES)] = (
                    mval + sval * eval_)

        def group(g, carry):
            idxvec = idx_v[pl.ds(g * _LANES, _LANES)]
            issue(idxvec[0], 0)
            for l in range(_LANES):
                if l + 1 < _LANES:
                    issue(idxvec[l + 1], (l + 1) % 2)
                drain(l % 2)
                extract(idxvec[l], g * _LANES + l, l, l % 2)
            pltpu.sync_copy(
                rows_v, out_hbm.at[pl.ds(base + g * _LANES, _LANES), :])
            return carry

        lax.fori_loop(0, b_per_w // _LANES, group, 0)

    return sampler


def kernel(y, mu, sigma):
    B = y.shape[0]
    V, D = mu.shape
    tail = V % _TW
    eps = jax.random.normal(jax.random.key(42), (B, D), dtype=mu.dtype)
    return _build_sampler(B, V, D)(
        y, mu.T, sigma.T, eps.T, mu[V - tail:].T, sigma[V - tail:].T)


# speedup vs baseline: 20.7355x; 1.1413x over previous
import functools

import jax
import jax.numpy as jnp
from jax import lax
from jax.experimental import pallas as pl
from jax.experimental.pallas import tpu as pltpu
from jax.experimental.pallas import tpu_sc as plsc

_LANES = 16
_TW = 128


@functools.cache
def _build_sampler(B, V, D):
    info = plsc.get_sparse_core_info()
    nc, ns = info.num_cores, info.num_subcores
    nw = nc * ns
    assert B % (8 * nw) == 0 and D % _LANES == 0
    b_per_w = B // nw
    tail = V % _TW
    vmain = V - tail
    tc_max = vmain - _TW
    mesh = plsc.VectorSubcoreMesh(core_axis_name="c", subcore_axis_name="s")

    @functools.partial(
        pl.kernel,
        mesh=mesh,
        out_type=jax.ShapeDtypeStruct((B, D), jnp.float32),
        compiler_params=pltpu.CompilerParams(needs_layout_passes=False),
        scratch_types=[
            pltpu.VMEM((b_per_w,), jnp.int32),
            pltpu.VMEM((D, b_per_w), jnp.float32),
            pltpu.VMEM((2, D, _TW), jnp.float32),
            pltpu.VMEM((2, D, _TW), jnp.float32),
            pltpu.VMEM((D, V % _TW), jnp.float32),
            pltpu.VMEM((D, V % _TW), jnp.float32),
            pltpu.VMEM((_LANES, D), jnp.float32),
            pltpu.SemaphoreType.DMA,
            pltpu.SemaphoreType.DMA,
            pltpu.SemaphoreType.DMA,
            pltpu.SemaphoreType.DMA,
        ],
    )
    def sampler(y_hbm, muT_hbm, sgT_hbm, epsT_hbm, tmu_hbm, tsg_hbm,
                out_hbm, idx_v, ep_v, mub_v, sgb_v, tmu_v, tsg_v, rows_v,
                sem_mu0, sem_mu1, sem_sg0, sem_sg1):
        sems = ((sem_mu0, sem_sg0), (sem_mu1, sem_sg1))
        wid = lax.axis_index("s") * nc + lax.axis_index("c")
        base = wid * b_per_w

        pltpu.sync_copy(y_hbm.at[pl.ds(base, b_per_w)], idx_v)
        pltpu.sync_copy(epsT_hbm.at[:, pl.ds(base, b_per_w)], ep_v)
        pltpu.sync_copy(tmu_hbm, tmu_v)
        pltpu.sync_copy(tsg_hbm, tsg_v)

        iotas = [
            lax.iota(jnp.int32, _LANES) + k * _LANES
            for k in range(D // _LANES)
        ]

        def issue(r, slot):
            tc = pl.multiple_of(
                jnp.minimum(r // _TW, tc_max // _TW) * _TW, _TW)
            pltpu.async_copy(
                muT_hbm.at[:, pl.ds(tc, _TW)], mub_v.at[slot], sems[slot][0])
            pltpu.async_copy(
                sgT_hbm.at[:, pl.ds(tc, _TW)], sgb_v.at[slot], sems[slot][1])

        def drain(slot):
            pltpu.make_async_copy(
                muT_hbm.at[:, pl.ds(0, _TW)], mub_v.at[slot],
                sems[slot][0]).wait()
            pltpu.make_async_copy(
                sgT_hbm.at[:, pl.ds(0, _TW)], sgb_v.at[slot],
                sems[slot][1]).wait()

        def extract(r, i, row, slot):
            tc = jnp.minimum(r // _TW, tc_max // _TW) * _TW
            rmod = jnp.minimum(r - tc, _TW - 1)
            use_tail = r >= vmain
            jt = jnp.clip(r - vmain, 0, tail - 1)
            jv = jnp.full((_LANES,), rmod, dtype=jnp.int32)
            jtv = jnp.full((_LANES,), jt, dtype=jnp.int32)
            pv = jnp.full((_LANES,), i, dtype=jnp.int32)
            for k in range(D // _LANES):
                cidx = iotas[k]
                mval = plsc.load_gather(mub_v.at[slot], [cidx, jv])
                sval = plsc.load_gather(sgb_v.at[slot], [cidx, jv])
                mtail = plsc.load_gather(tmu_v, [cidx, jtv])
                stail = plsc.load_gather(tsg_v, [cidx, jtv])
                mval = jnp.where(use_tail, mtail, mval)
                sval = jnp.where(use_tail, stail, sval)
                eval_ = plsc.load_gather(ep_v, [cidx, pv])
                rows_v[row, pl.ds(k * _LANES, _LANES)] = (
                    mval + sval * eval_)

        def group(g, tc_cur):
            idxvec = idx_v[pl.ds(g * _LANES, _LANES)]
            for l in range(_LANES):
                r = idxvec[l]
                tc = jnp.minimum(r // _TW, tc_max // _TW) * _TW

                @pl.when(tc != tc_cur)
                def _():
                    issue(r, 0)
                    drain(0)

                extract(r, g * _LANES + l, l, 0)
                tc_cur = tc
            pltpu.sync_copy(
                rows_v, out_hbm.at[pl.ds(base + g * _LANES, _LANES), :])
            return tc_cur

        lax.fori_loop(0, b_per_w // _LANES, group,
                      jnp.int32(-1))

    return sampler


def kernel(y, mu, sigma):
    B = y.shape[0]
    V, D = mu.shape
    tail = V % _TW
    eps = jax.random.normal(jax.random.key(42), (B, D), dtype=mu.dtype)
    iota = jnp.arange(B, dtype=jnp.int32)
    ys, perm = lax.sort_key_val(y, iota)
    eps_s = jnp.take(eps, perm, axis=0)
    out_s = _build_sampler(B, V, D)(
        ys, mu.T, sigma.T, eps_s.T, mu[V - tail:].T, sigma[V - tail:].T)
    inv = jnp.zeros_like(iota).at[perm].set(iota)
    return jnp.take(out_s, inv, axis=0)


# speedup vs baseline: 23.5201x; 1.1343x over previous
import functools

import jax
import jax.numpy as jnp
from jax import lax
from jax.experimental import pallas as pl
from jax.experimental.pallas import tpu as pltpu
from jax.experimental.pallas import tpu_sc as plsc

_LANES = 16
_TW = 128
_NSUB = 4


@functools.cache
def _build_sampler(B, V, D):
    info = plsc.get_sparse_core_info()
    nc, ns = info.num_cores, info.num_subcores
    nw = nc * ns
    assert B % (8 * nw) == 0 and D % _LANES == 0
    b_per_w = B // nw
    tail = V % _TW
    vmain = V - tail
    tc_max = vmain - _TW
    mesh = plsc.VectorSubcoreMesh(core_axis_name="c", subcore_axis_name="s")

    @functools.partial(
        pl.kernel,
        mesh=mesh,
        out_type=jax.ShapeDtypeStruct((B, D), jnp.float32),
        compiler_params=pltpu.CompilerParams(needs_layout_passes=False),
        scratch_types=[
            pltpu.VMEM((b_per_w,), jnp.int32),
            pltpu.VMEM((D, b_per_w), jnp.float32),
            pltpu.VMEM((_NSUB, D, _TW), jnp.float32),
            pltpu.VMEM((_NSUB, D, _TW), jnp.float32),
            pltpu.VMEM((D, V % _TW), jnp.float32),
            pltpu.VMEM((D, V % _TW), jnp.float32),
            pltpu.VMEM((_NSUB, _LANES, D), jnp.float32),
        ] + [pltpu.SemaphoreType.DMA] * (2 * _NSUB),
    )
    def sampler(y_hbm, muT_hbm, sgT_hbm, epsT_hbm, tmu_hbm, tsg_hbm,
                out_hbm, idx_v, ep_v, mub_v, sgb_v, tmu_v, tsg_v, rows_v,
                *semlist):
        sems = tuple(
            (semlist[2 * s], semlist[2 * s + 1]) for s in range(_NSUB))
        wid = lax.axis_index("s") * nc + lax.axis_index("c")
        base = wid * b_per_w

        pltpu.sync_copy(y_hbm.at[pl.ds(base, b_per_w)], idx_v)
        pltpu.sync_copy(epsT_hbm.at[:, pl.ds(base, b_per_w)], ep_v)
        pltpu.sync_copy(tmu_hbm, tmu_v)
        pltpu.sync_copy(tsg_hbm, tsg_v)

        iotas = [
            lax.iota(jnp.int32, _LANES) + k * _LANES
            for k in range(D // _LANES)
        ]

        def issue(r, slot):
            tc = pl.multiple_of(
                jnp.minimum(r // _TW, tc_max // _TW) * _TW, _TW)
            pltpu.async_copy(
                muT_hbm.at[:, pl.ds(tc, _TW)], mub_v.at[slot], sems[slot][0])
            pltpu.async_copy(
                sgT_hbm.at[:, pl.ds(tc, _TW)], sgb_v.at[slot], sems[slot][1])

        def drain(slot):
            pltpu.make_async_copy(
                muT_hbm.at[:, pl.ds(0, _TW)], mub_v.at[slot],
                sems[slot][0]).wait()
            pltpu.make_async_copy(
                sgT_hbm.at[:, pl.ds(0, _TW)], sgb_v.at[slot],
                sems[slot][1]).wait()

        def extract(r, i, row, slot):
            tc = jnp.minimum(r // _TW, tc_max // _TW) * _TW
            rmod = jnp.minimum(r - tc, _TW - 1)
            use_tail = r >= vmain
            jt = jnp.clip(r - vmain, 0, tail - 1)
            jv = jnp.full((_LANES,), rmod, dtype=jnp.int32)
            jtv = jnp.full((_LANES,), jt, dtype=jnp.int32)
            pv = jnp.full((_LANES,), i, dtype=jnp.int32)
            for k in range(D // _LANES):
                cidx = iotas[k]
                mval = plsc.load_gather(mub_v.at[slot], [cidx, jv])
                sval = plsc.load_gather(sgb_v.at[slot], [cidx, jv])
                mtail = plsc.load_gather(tmu_v, [cidx, jtv])
                stail = plsc.load_gather(tsg_v, [cidx, jtv])
                mval = jnp.where(use_tail, mtail, mval)
                sval = jnp.where(use_tail, stail, sval)
                eval_ = plsc.load_gather(ep_v, [cidx, pv])
                rows_v[slot, row, pl.ds(k * _LANES, _LANES)] = (
                    mval + sval * eval_)

        gps = b_per_w // _LANES // _NSUB

        def round_(r0, tcs):
            tcs = list(tcs)
            idxvecs = [
                idx_v[pl.ds((gps * s + r0) * _LANES, _LANES)]
                for s in range(_NSUB)
            ]
            for l in range(_LANES):
                conds, rs = [], []
                for s in range(_NSUB):
                    r = idxvecs[s][l]
                    tc = jnp.minimum(r // _TW, tc_max // _TW) * _TW
                    cond = tc != tcs[s]
                    conds.append(cond)
                    rs.append(r)
                    tcs[s] = tc

                    @pl.when(cond)
                    def _(r=r, s=s):
                        issue(r, s)

                for s in range(_NSUB):
                    @pl.when(conds[s])
                    def _(s=s):
                        drain(s)

                    extract(rs[s], (gps * s + r0) * _LANES + l, l, s)

            for s in range(_NSUB):
                pltpu.sync_copy(
                    rows_v.at[s],
                    out_hbm.at[pl.ds(base + (gps * s + r0) * _LANES,
                                     _LANES), :])
            return tuple(tcs)

        lax.fori_loop(0, gps, round_,
                      tuple(jnp.int32(-1) for _ in range(_NSUB)))

    return sampler


def kernel(y, mu, sigma):
    B = y.shape[0]
    V, D = mu.shape
    tail = V % _TW
    eps = jax.random.normal(jax.random.key(42), (B, D), dtype=mu.dtype)
    iota = jnp.arange(B, dtype=jnp.int32)
    ys, perm = lax.sort_key_val(y, iota)
    eps_s = jnp.take(eps, perm, axis=0)
    out_s = _build_sampler(B, V, D)(
        ys, mu.T, sigma.T, eps_s.T, mu[V - tail:].T, sigma[V - tail:].T)
    inv = jnp.zeros_like(iota).at[perm].set(iota)
    return jnp.take(out_s, inv, axis=0)


# speedup vs baseline: 23.5581x; 1.0016x over previous
import functools

import jax
import jax.numpy as jnp
from jax import lax
from jax.experimental import pallas as pl
from jax.experimental.pallas import tpu as pltpu
from jax.experimental.pallas import tpu_sc as plsc

_LANES = 16
_TW = 128
_NSUB = 4


@functools.cache
def _build_sampler(B, V, D):
    info = plsc.get_sparse_core_info()
    nc, ns = info.num_cores, info.num_subcores
    nw = nc * ns
    assert B % (8 * nw) == 0 and D % _LANES == 0
    b_per_w = B // nw
    tail = V % _TW
    vmain = V - tail
    tc_max = vmain - _TW
    mesh = plsc.VectorSubcoreMesh(core_axis_name="c", subcore_axis_name="s")

    @functools.partial(
        pl.kernel,
        mesh=mesh,
        out_type=jax.ShapeDtypeStruct((B, D), jnp.float32),
        compiler_params=pltpu.CompilerParams(needs_layout_passes=False),
        scratch_types=[
            pltpu.VMEM((b_per_w,), jnp.int32),
            pltpu.VMEM((D, b_per_w), jnp.float32),
            pltpu.VMEM((_NSUB, D, _TW), jnp.float32),
            pltpu.VMEM((_NSUB, D, _TW), jnp.float32),
            pltpu.VMEM((D, V % _TW), jnp.float32),
            pltpu.VMEM((D, V % _TW), jnp.float32),
            pltpu.VMEM((_NSUB, _LANES, D), jnp.float32),
        ] + [pltpu.SemaphoreType.DMA] * (2 * _NSUB),
    )
    def sampler(y_hbm, muT_hbm, sgT_hbm, epsT_hbm, tmu_hbm, tsg_hbm,
                out_hbm, idx_v, ep_v, mub_v, sgb_v, tmu_v, tsg_v, rows_v,
                *semlist):
        sems = tuple(
            (semlist[2 * s], semlist[2 * s + 1]) for s in range(_NSUB))
        wid = lax.axis_index("s") * nc + lax.axis_index("c")
        base = wid * b_per_w

        pltpu.sync_copy(y_hbm.at[pl.ds(base, b_per_w)], idx_v)
        pltpu.sync_copy(epsT_hbm.at[:, pl.ds(base, b_per_w)], ep_v)
        pltpu.sync_copy(tmu_hbm, tmu_v)
        pltpu.sync_copy(tsg_hbm, tsg_v)

        iotas = [
            lax.iota(jnp.int32, _LANES) + k * _LANES
            for k in range(D // _LANES)
        ]

        def issue(r, slot):
            tc = pl.multiple_of(
                jnp.minimum(r // _TW, tc_max // _TW) * _TW, _TW)
            pltpu.async_copy(
                muT_hbm.at[:, pl.ds(tc, _TW)], mub_v.at[slot], sems[slot][0])
            pltpu.async_copy(
                sgT_hbm.at[:, pl.ds(tc, _TW)], sgb_v.at[slot], sems[slot][1])

        def drain(slot):
            pltpu.make_async_copy(
                muT_hbm.at[:, pl.ds(0, _TW)], mub_v.at[slot],
                sems[slot][0]).wait()
            pltpu.make_async_copy(
                sgT_hbm.at[:, pl.ds(0, _TW)], sgb_v.at[slot],
                sems[slot][1]).wait()

        def extract(r, i, row, slot):
            tc = jnp.minimum(r // _TW, tc_max // _TW) * _TW
            rmod = jnp.minimum(r - tc, _TW - 1)
            use_tail = r >= vmain
            jt = jnp.clip(r - vmain, 0, tail - 1)
            jv = jnp.full((_LANES,), rmod, dtype=jnp.int32)
            jtv = jnp.full((_LANES,), jt, dtype=jnp.int32)
            pv = jnp.full((_LANES,), i, dtype=jnp.int32)
            for k in range(D // _LANES):
                cidx = iotas[k]
                mval = plsc.load_gather(mub_v.at[slot], [cidx, jv])
                sval = plsc.load_gather(sgb_v.at[slot], [cidx, jv])
                mtail = plsc.load_gather(tmu_v, [cidx, jtv])
                stail = plsc.load_gather(tsg_v, [cidx, jtv])
                mval = jnp.where(use_tail, mtail, mval)
                sval = jnp.where(use_tail, stail, sval)
                eval_ = plsc.load_gather(ep_v, [cidx, pv])
                rows_v[slot, row, pl.ds(k * _LANES, _LANES)] = (
                    mval + sval * eval_)

        gps = b_per_w // _LANES // _NSUB

        def round_(r0, tcs):
            tcs = list(tcs)
            idxvecs = [
                idx_v[pl.ds((gps * s + r0) * _LANES, _LANES)]
                for s in range(_NSUB)
            ]
            for l in range(_LANES):
                conds, rs = [], []
                for s in range(_NSUB):
                    r = idxvecs[s][l]
                    tc = jnp.minimum(r // _TW, tc_max // _TW) * _TW
                    cond = tc != tcs[s]
                    conds.append(cond)
                    rs.append(r)
                    tcs[s] = tc

                    @pl.when(cond)
                    def _(r=r, s=s):
                        issue(r, s)

                for s in range(_NSUB):
                    @pl.when(conds[s])
                    def _(s=s):
                        drain(s)

                    extract(rs[s], (gps * s + r0) * _LANES + l, l, s)

            for s in range(_NSUB):
                pltpu.sync_copy(
                    rows_v.at[s],
                    out_hbm.at[pl.ds(base + (gps * s + r0) * _LANES,
                                     _LANES), :])
            return tuple(tcs)

        lax.fori_loop(0, gps, round_,
                      tuple(jnp.int32(-1) for _ in range(_NSUB)))

    return sampler


def kernel(y, mu, sigma):
    B = y.shape[0]
    V, D = mu.shape
    tail = V % _TW
    eps = jax.random.normal(jax.random.key(42), (B, D), dtype=mu.dtype)
    iota = jnp.arange(B, dtype=jnp.int32)
    packed = jnp.sort((y // _TW).astype(jnp.uint32) * jnp.uint32(B)
                      + iota.astype(jnp.uint32))
    perm = (packed % jnp.uint32(B)).astype(jnp.int32)
    ys = jnp.take(y, perm, axis=0)
    eps_s = jnp.take(eps, perm, axis=0)
    out_s = _build_sampler(B, V, D)(
        ys, mu.T, sigma.T, eps_s.T, mu[V - tail:].T, sigma[V - tail:].T)
    inv = jnp.zeros_like(iota).at[perm].set(iota)
    return jnp.take(out_s, inv, axis=0)


# speedup vs baseline: 23.5611x; 1.0001x over previous
import functools

import jax
import jax.numpy as jnp
from jax import lax
from jax.experimental import pallas as pl
from jax.experimental.pallas import tpu as pltpu
from jax.experimental.pallas import tpu_sc as plsc

_LANES = 16
_TW = 128
_NSUB = 4


@functools.cache
def _build_sampler(B, V, D):
    info = plsc.get_sparse_core_info()
    nc, ns = info.num_cores, info.num_subcores
    nw = nc * ns
    assert B % (8 * nw) == 0 and D % _LANES == 0
    b_per_w = B // nw
    tail = V % _TW
    vmain = V - tail
    tc_max = vmain - _TW
    mesh = plsc.VectorSubcoreMesh(core_axis_name="c", subcore_axis_name="s")

    @functools.partial(
        pl.kernel,
        mesh=mesh,
        out_type=jax.ShapeDtypeStruct((B, D), jnp.float32),
        compiler_params=pltpu.CompilerParams(needs_layout_passes=False),
        scratch_types=[
            pltpu.VMEM((b_per_w,), jnp.int32),
            pltpu.VMEM((D, b_per_w), jnp.float32),
            pltpu.VMEM((_NSUB, D, _TW), jnp.float32),
            pltpu.VMEM((_NSUB, D, _TW), jnp.float32),
            pltpu.VMEM((D, V % _TW), jnp.float32),
            pltpu.VMEM((D, V % _TW), jnp.float32),
            pltpu.VMEM((_NSUB, _LANES, D), jnp.float32),
        ] + [pltpu.SemaphoreType.DMA] * (3 * _NSUB),
    )
    def sampler(y_hbm, muT_hbm, sgT_hbm, epsT_hbm, tmu_hbm, tsg_hbm,
                out_hbm, idx_v, ep_v, mub_v, sgb_v, tmu_v, tsg_v, rows_v,
                *semlist):
        sems = tuple(
            (semlist[2 * s], semlist[2 * s + 1]) for s in range(_NSUB))
        osems = semlist[2 * _NSUB:]
        wid = lax.axis_index("s") * nc + lax.axis_index("c")
        base = wid * b_per_w

        pltpu.sync_copy(y_hbm.at[pl.ds(base, b_per_w)], idx_v)
        pltpu.sync_copy(epsT_hbm.at[:, pl.ds(base, b_per_w)], ep_v)
        pltpu.sync_copy(tmu_hbm, tmu_v)
        pltpu.sync_copy(tsg_hbm, tsg_v)

        iotas = [
            lax.iota(jnp.int32, _LANES) + k * _LANES
            for k in range(D // _LANES)
        ]

        def issue(r, slot):
            tc = pl.multiple_of(
                jnp.minimum(r // _TW, tc_max // _TW) * _TW, _TW)
            pltpu.async_copy(
                muT_hbm.at[:, pl.ds(tc, _TW)], mub_v.at[slot], sems[slot][0])
            pltpu.async_copy(
                sgT_hbm.at[:, pl.ds(tc, _TW)], sgb_v.at[slot], sems[slot][1])

        def drain(slot):
            pltpu.make_async_copy(
                muT_hbm.at[:, pl.ds(0, _TW)], mub_v.at[slot],
                sems[slot][0]).wait()
            pltpu.make_async_copy(
                sgT_hbm.at[:, pl.ds(0, _TW)], sgb_v.at[slot],
                sems[slot][1]).wait()

        def extract(r, i, row, slot):
            tc = jnp.minimum(r // _TW, tc_max // _TW) * _TW
            rmod = jnp.minimum(r - tc, _TW - 1)
            use_tail = r >= vmain
            jt = jnp.clip(r - vmain, 0, tail - 1)
            jv = jnp.full((_LANES,), rmod, dtype=jnp.int32)
            jtv = jnp.full((_LANES,), jt, dtype=jnp.int32)
            pv = jnp.full((_LANES,), i, dtype=jnp.int32)
            for k in range(D // _LANES):
                cidx = iotas[k]
                mval = plsc.load_gather(mub_v.at[slot], [cidx, jv])
                sval = plsc.load_gather(sgb_v.at[slot], [cidx, jv])
                mtail = plsc.load_gather(tmu_v, [cidx, jtv])
                stail = plsc.load_gather(tsg_v, [cidx, jtv])
                mval = jnp.where(use_tail, mtail, mval)
                sval = jnp.where(use_tail, stail, sval)
                eval_ = plsc.load_gather(ep_v, [cidx, pv])
                rows_v[slot, row, pl.ds(k * _LANES, _LANES)] = (
                    mval + sval * eval_)

        gps = b_per_w // _LANES // _NSUB

        def out_slab(s, r0):
            return out_hbm.at[pl.ds(base + (gps * s + r0) * _LANES,
                                    _LANES), :]

        def round_(r0, tcs):
            tcs = list(tcs)
            idxvecs = [
                idx_v[pl.ds((gps * s + r0) * _LANES, _LANES)]
                for s in range(_NSUB)
            ]

            @pl.when(r0 > 0)
            def _():
                for s in range(_NSUB):
                    pltpu.make_async_copy(
                        rows_v.at[s], out_slab(s, 0), osems[s]).wait()
            for l in range(_LANES):
                conds, rs = [], []
                for s in range(_NSUB):
                    r = idxvecs[s][l]
                    tc = jnp.minimum(r // _TW, tc_max // _TW) * _TW
                    cond = tc != tcs[s]
                    conds.append(cond)
                    rs.append(r)
                    tcs[s] = tc

                    @pl.when(cond)
                    def _(r=r, s=s):
                        issue(r, s)

                for s in range(_NSUB):
                    @pl.when(conds[s])
                    def _(s=s):
                        drain(s)

                    extract(rs[s], (gps * s + r0) * _LANES + l, l, s)

            for s in range(_NSUB):
                pltpu.async_copy(rows_v.at[s], out_slab(s, r0), osems[s])
            return tuple(tcs)

        lax.fori_loop(0, gps, round_,
                      tuple(jnp.int32(-1) for _ in range(_NSUB)))
        for s in range(_NSUB):
            pltpu.make_async_copy(
                rows_v.at[s], out_slab(s, 0), osems[s]).wait()

    return sampler


def kernel(y, mu, sigma):
    B = y.shape[0]
    V, D = mu.shape
    tail = V % _TW
    eps = jax.random.normal(jax.random.key(42), (B, D), dtype=mu.dtype)
    iota = jnp.arange(B, dtype=jnp.int32)
    packed = jnp.sort((y // _TW).astype(jnp.uint32) * jnp.uint32(B)
                      + iota.astype(jnp.uint32))
    perm = (packed % jnp.uint32(B)).astype(jnp.int32)
    ys = jnp.take(y, perm, axis=0)
    eps_s = jnp.take(eps, perm, axis=0)
    out_s = _build_sampler(B, V, D)(
        ys, mu.T, sigma.T, eps_s.T, mu[V - tail:].T, sigma[V - tail:].T)
    inv = jnp.zeros_like(iota).at[perm].set(iota)
    return jnp.take(out_s, inv, axis=0)
